# Initial kernel scaffold; baseline (speedup 1.0000x reference)
import jax
import jax.numpy as jnp
from jax.experimental import pallas as pl


def kernel(x, edge_index, edge_label_index, emb, W1, b1, W2, b2):
    raise NotImplementedError("write your pallas kernel here")


# speedup vs baseline: 4.1362x; 4.1362x over previous
import functools

import jax
import jax.numpy as jnp
from jax import lax
from jax.experimental import pallas as pl
from jax.experimental.pallas import tpu as pltpu
from jax.experimental.pallas import tpu_sc as plsc

N = 10000
NP = 10240
E = 320000
NLP = 40000
NLP_PAD = 40960
D0 = 128
D2 = 64
NW = 16
CE = 128
KE = 157
E_PAD = NW * KE * CE
KX = NP // NW // CE
KL = NLP_PAD // NW // CE
RPT = NP // NW
DW = 16
PAD_DST = NP - 1

_MESH = plsc.VectorSubcoreMesh(core_axis_name="c", subcore_axis_name="s",
                               num_cores=1)
_SC_PARAMS = pltpu.CompilerParams(use_tc_tiling_on_sc=False,
                                  needs_layout_passes=False)


def _f32(shape):
    return jax.ShapeDtypeStruct(shape, jnp.float32)


@functools.partial(
    pl.kernel,
    out_type=(_f32((NP, D0)), _f32((NP // 8, D0))),
    mesh=_MESH,
    compiler_params=_SC_PARAMS,
    scratch_types=[
        pltpu.VMEM((KX, CE), jnp.int32),
        pltpu.VMEM((KE, CE), jnp.int32),
        pltpu.VMEM((CE, D0), jnp.float32),
        pltpu.VMEM((CE, DW), jnp.float32),
        pltpu.VMEM((RPT, DW), jnp.float32),
        pltpu.VMEM((RPT // 8, D0), jnp.float32),
        pltpu.VMEM_SHARED((NP, DW), jnp.float32),
        pltpu.SemaphoreType.DMA,
    ],
)
def _sc_gather_deg(emb_hbm, xi_hbm, dst_hbm, h0_hbm, degt_hbm,
                   xidx_v, didx_v, rows_v, ones_v, stga_v, stgb_v,
                   deg_sh, sem):
    s = lax.axis_index("s")
    w = s

    def _zero_row(i, _):
        stga_v[i, :] = jnp.zeros((DW,), jnp.float32)
        return _
    lax.fori_loop(0, RPT, _zero_row, None)
    pltpu.sync_copy(stga_v, deg_sh.at[pl.ds(s * RPT, RPT)])

    pltpu.sync_copy(xi_hbm.at[w], xidx_v)
    for k in range(KX):
        pltpu.async_copy(emb_hbm.at[xidx_v.at[k]], rows_v, sem).wait()
        pltpu.sync_copy(rows_v, h0_hbm.at[pl.ds((w * KX + k) * CE, CE)])

    def _ones_row(i, _):
        ones_v[i, :] = jnp.full((DW,), 1.0, jnp.float32)
        return _
    lax.fori_loop(0, CE, _ones_row, None)

    plsc.subcore_barrier()

    pltpu.sync_copy(dst_hbm.at[w], didx_v)

    def _deg_chunk(j, _):
        pltpu.sync_copy(ones_v, deg_sh.at[didx_v.at[j]], add=True)
        return _
    lax.fori_loop(0, KE, _deg_chunk, None)

    plsc.subcore_barrier()

    pltpu.sync_copy(deg_sh.at[pl.ds(s * RPT, RPT)], stga_v)

    def _repack(i, _):
        for m in range(8):
            stgb_v[i, pl.ds(m * DW, DW)] = stga_v[i * 8 + m, :]
        return _
    lax.fori_loop(0, RPT // 8, _repack, None)
    pltpu.sync_copy(stgb_v, degt_hbm.at[pl.ds(s * (RPT // 8), RPT // 8)])


@functools.partial(
    pl.kernel,
    out_type=_f32((2, NP // 2, D0)),
    mesh=_MESH,
    compiler_params=_SC_PARAMS,
    scratch_types=[
        pltpu.VMEM((KE, CE), jnp.int32),
        pltpu.VMEM((KE, CE), jnp.int32),
        pltpu.VMEM((16,), jnp.int32),
        pltpu.VMEM((CE, D0), jnp.float32),
        pltpu.VMEM((CE, D2), jnp.float32),
        pltpu.VMEM((160, D2), jnp.float32),
        pltpu.VMEM((80, D0), jnp.float32),
        pltpu.VMEM_SHARED((NP, D2), jnp.float32),
        pltpu.SemaphoreType.DMA,
    ],
)
def _sc_edge_pass(y_hbm, src_hbm, dst_hbm, nch_hbm, aggp_hbm,
                  sidx_v, didx_v, nch_v, rows_v, half_v, stga_v, stgb_v,
                  acc_sh, sem):
    s = lax.axis_index("s")
    w = s

    pltpu.sync_copy(src_hbm.at[w], sidx_v)
    pltpu.sync_copy(dst_hbm.at[w], didx_v)
    pltpu.sync_copy(nch_hbm, nch_v)
    nch_vec = nch_v[...]

    def _zero_row(i, _):
        for q in range(D2 // 16):
            stga_v[i, pl.ds(q * 16, 16)] = jnp.zeros((16,), jnp.float32)
        return _

    for h in (0, 1):
        lax.fori_loop(0, 160, _zero_row, None)
        for k in range(4):
            pltpu.sync_copy(stga_v, acc_sh.at[pl.ds(s * RPT + k * 160, 160)])
        plsc.subcore_barrier()

        def _chunk(j, _):
            pltpu.async_copy(y_hbm.at[sidx_v.at[j]], rows_v, sem).wait()

            def _extract(r, __):
                for q in range(D2 // 16):
                    half_v[r, pl.ds(q * 16, 16)] = (
                        rows_v[r, pl.ds(h * D2 + q * 16, 16)])
                return __
            lax.fori_loop(0, CE, _extract, None)
            pltpu.sync_copy(half_v, acc_sh.at[didx_v.at[j]], add=True)
            return _
        lax.fori_loop(0, nch_vec[h], _chunk, None)

        plsc.subcore_barrier()

        for k in range(4):
            pltpu.sync_copy(acc_sh.at[pl.ds(s * RPT + k * 160, 160)], stga_v)

            def _repack(i, _):
                for m in range(2):
                    for q in range(D2 // 16):
                        stgb_v[i, pl.ds(m * D2 + q * 16, 16)] = (
                            stga_v[i * 2 + m, pl.ds(q * 16, 16)])
                return _
            lax.fori_loop(0, 80, _repack, None)
            pltpu.sync_copy(
                stgb_v, aggp_hbm.at[h, pl.ds(s * (RPT // 2) + k * 80, 80)])


@functools.partial(
    pl.kernel,
    out_type=_f32((NW, KL, CE)),
    mesh=_MESH,
    compiler_params=_SC_PARAMS,
    scratch_types=[
        pltpu.VMEM((KL, CE), jnp.int32),
        pltpu.VMEM((KL, CE), jnp.int32),
        pltpu.VMEM((CE, D0), jnp.float32),
        pltpu.VMEM((CE, D0), jnp.float32),
        pltpu.VMEM((KL, CE), jnp.float32),
        pltpu.SemaphoreType.DMA,
    ],
)
def _sc_decode(z_hbm, si_hbm, di_hbm, res_hbm,
               sidx_v, didx_v, srows_v, drows_v, out_v, sem):
    s = lax.axis_index("s")
    w = s

    pltpu.sync_copy(si_hbm.at[w], sidx_v)
    pltpu.sync_copy(di_hbm.at[w], didx_v)

    def _chunk(t, _):
        pltpu.async_copy(z_hbm.at[sidx_v.at[t]], srows_v, sem).wait()
        pltpu.async_copy(z_hbm.at[didx_v.at[t]], drows_v, sem).wait()
        lane = lax.iota(jnp.int32, 16)
        for g in range(CE // 16):
            erow = lane + g * 16
            acc = jnp.zeros((16,), jnp.float32)
            for j in range(D2):
                col = jnp.full((16,), j, jnp.int32)
                sv = plsc.load_gather(srows_v, [erow, col])
                dv = plsc.load_gather(drows_v, [erow, col])
                acc = acc + sv * dv
            out_v[t, pl.ds(g * 16, 16)] = acc
        return _
    lax.fori_loop(0, KL, _chunk, None)
    pltpu.sync_copy(out_v, res_hbm.at[w])


_TCB = 1280
_TCG = NP // _TCB


def _tc1_body(h0_ref, w1_ref, degt_ref, xw_ref, y_ref, dinv_ref):
    deg = degt_ref[:, 0:1] + 1.0
    dinv = lax.rsqrt(deg)
    xw = jnp.dot(h0_ref[...], w1_ref[...], preferred_element_type=jnp.float32)
    xw_ref[...] = xw
    y_ref[...] = xw * dinv
    dinv_ref[...] = dinv


def _tc2_body(aggp_ref, xw1_ref, dinv_ref, w2_ref, b1_ref,
              xw2_ref, y2_ref):
    dinv = dinv_ref[...]
    agg = jnp.concatenate([aggp_ref[0], aggp_ref[1]], axis=1)
    h1 = jnp.maximum(agg * dinv + xw1_ref[...] * (dinv * dinv)
                     + b1_ref[...][None, :], 0.0)
    xw2 = jnp.dot(h1, w2_ref[...], preferred_element_type=jnp.float32)
    xw2_ref[...] = xw2
    y2 = xw2 * dinv
    y2_ref[...] = jnp.concatenate(
        [y2, jnp.zeros_like(y2)], axis=1)


def _tc3_body(aggp_ref, xw2_ref, dinv_ref, b2_ref, z_ref):
    dinv = dinv_ref[...]
    agg = aggp_ref[0]
    z = (agg * dinv + xw2_ref[...] * (dinv * dinv)
         + b2_ref[...][None, :])
    z_ref[...] = jnp.concatenate(
        [z, jnp.zeros_like(z)], axis=1)


def _tc1(h0, W1, degt):
    return pl.pallas_call(
        _tc1_body,
        grid=(_TCG,),
        in_specs=[
            pl.BlockSpec((_TCB, D0), lambda i: (i, 0)),
            pl.BlockSpec((D0, D0), lambda i: (0, 0)),
            pl.BlockSpec((_TCB, DW), lambda i: (i, 0)),
        ],
        out_specs=[
            pl.BlockSpec((_TCB, D0), lambda i: (i, 0)),
            pl.BlockSpec((_TCB, D0), lambda i: (i, 0)),
            pl.BlockSpec((_TCB, 1), lambda i: (i, 0)),
        ],
        out_shape=[_f32((NP, D0)), _f32((NP, D0)), _f32((NP, 1))],
    )(h0, W1, degt)


def _tc2(aggp1, xw1, dinv, W2, b1):
    return pl.pallas_call(
        _tc2_body,
        grid=(_TCG,),
        in_specs=[
            pl.BlockSpec((2, _TCB, D2), lambda i: (0, i, 0)),
            pl.BlockSpec((_TCB, D0), lambda i: (i, 0)),
            pl.BlockSpec((_TCB, 1), lambda i: (i, 0)),
            pl.BlockSpec((D0, D2), lambda i: (0, 0)),
            pl.BlockSpec((D0,), lambda i: (0,)),
        ],
        out_specs=[
            pl.BlockSpec((_TCB, D2), lambda i: (i, 0)),
            pl.BlockSpec((_TCB, D0), lambda i: (i, 0)),
        ],
        out_shape=[_f32((NP, D2)), _f32((NP, D0))],
    )(aggp1, xw1, dinv, W2, b1)


def _tc3(aggp2, xw2, dinv, b2):
    return pl.pallas_call(
        _tc3_body,
        grid=(_TCG,),
        in_specs=[
            pl.BlockSpec((2, _TCB, D2), lambda i: (0, i, 0)),
            pl.BlockSpec((_TCB, D2), lambda i: (i, 0)),
            pl.BlockSpec((_TCB, 1), lambda i: (i, 0)),
            pl.BlockSpec((D2,), lambda i: (0,)),
        ],
        out_specs=pl.BlockSpec((_TCB, D0), lambda i: (i, 0)),
        out_shape=_f32((NP, D0)),
    )(aggp2, xw2, dinv, b2)


def kernel(x, edge_index, edge_label_index, emb, W1, b1, W2, b2):
    xi = jnp.pad(x[:, 0].astype(jnp.int32), (0, NP - N)).reshape(NW, KX, CE)
    src = jnp.pad(edge_index[0].astype(jnp.int32),
                  (0, E_PAD - E)).reshape(NW, KE, CE)
    dst = jnp.pad(edge_index[1].astype(jnp.int32), (0, E_PAD - E),
                  constant_values=PAD_DST).reshape(NW, KE, CE)
    si = jnp.pad(edge_label_index[0].astype(jnp.int32),
                 (0, NLP_PAD - NLP)).reshape(NW, KL, CE)
    di = jnp.pad(edge_label_index[1].astype(jnp.int32),
                 (0, NLP_PAD - NLP)).reshape(NW, KL, CE)
    nch1 = jnp.full((16,), KE, jnp.int32)
    nch2 = jnp.full((16,), KE, jnp.int32).at[1].set(0)

    h0, degt = _sc_gather_deg(emb, xi, dst)
    degt = degt.reshape(NP, DW)
    xw1, y1, dinv = _tc1(h0, W1, degt)
    aggp1 = _sc_edge_pass(y1, src, dst, nch1).reshape(2, NP, D2)
    xw2, y2 = _tc2(aggp1, xw1, dinv, W2, b1)
    aggp2 = _sc_edge_pass(y2, src, dst, nch2).reshape(2, NP, D2)
    z = _tc3(aggp2, xw2, dinv, b2)
    res = _sc_decode(z, si, di)
    return res.reshape(NLP_PAD)[:NLP]


# speedup vs baseline: 6.2327x; 1.5069x over previous
import functools

import jax
import jax.numpy as jnp
from jax import lax
from jax.experimental import pallas as pl
from jax.experimental.pallas import tpu as pltpu
from jax.experimental.pallas import tpu_sc as plsc

N = 10000
NP = 10240
E = 320000
NLP = 40000
NLP_PAD = 40960
D0 = 128
D2 = 64
NW = 16
CE = 128
KE = 157
E_PAD = NW * KE * CE
KX = NP // NW // CE
KL = NLP_PAD // NW // CE
RPT = NP // NW
DW = 16
PAD_DST = NP - 1

_MESH = plsc.VectorSubcoreMesh(core_axis_name="c", subcore_axis_name="s",
                               num_cores=1)
_SC_PARAMS = pltpu.CompilerParams(use_tc_tiling_on_sc=False,
                                  needs_layout_passes=False)


def _f32(shape):
    return jax.ShapeDtypeStruct(shape, jnp.float32)


@functools.partial(
    pl.kernel,
    out_type=(_f32((NP, D0)), _f32((NP // 8, D0))),
    mesh=_MESH,
    compiler_params=_SC_PARAMS,
    scratch_types=[
        pltpu.VMEM((KX, CE), jnp.int32),
        pltpu.VMEM((KE, CE), jnp.int32),
        pltpu.VMEM((CE, D0), jnp.float32),
        pltpu.VMEM((CE, DW), jnp.float32),
        pltpu.VMEM((RPT, DW), jnp.float32),
        pltpu.VMEM((RPT // 8, D0), jnp.float32),
        pltpu.VMEM_SHARED((NP, DW), jnp.float32),
        pltpu.SemaphoreType.DMA,
    ],
)
def _sc_gather_deg(emb_hbm, xi_hbm, dst_hbm, h0_hbm, degt_hbm,
                   xidx_v, didx_v, rows_v, ones_v, stga_v, stgb_v,
                   deg_sh, sem):
    s = lax.axis_index("s")
    w = s

    def _zero_row(i, _):
        stga_v[i, :] = jnp.zeros((DW,), jnp.float32)
        return _
    lax.fori_loop(0, RPT, _zero_row, None)
    pltpu.sync_copy(stga_v, deg_sh.at[pl.ds(s * RPT, RPT)])

    pltpu.sync_copy(xi_hbm.at[w], xidx_v)
    for k in range(KX):
        pltpu.async_copy(emb_hbm.at[xidx_v.at[k]], rows_v, sem).wait()
        pltpu.sync_copy(rows_v, h0_hbm.at[pl.ds((w * KX + k) * CE, CE)])

    def _ones_row(i, _):
        ones_v[i, :] = jnp.full((DW,), 1.0, jnp.float32)
        return _
    lax.fori_loop(0, CE, _ones_row, None)

    plsc.subcore_barrier()

    pltpu.sync_copy(dst_hbm.at[w], didx_v)

    def _deg_chunk(j, _):
        pltpu.sync_copy(ones_v, deg_sh.at[didx_v.at[j]], add=True)
        return _
    lax.fori_loop(0, KE, _deg_chunk, None)

    plsc.subcore_barrier()

    pltpu.sync_copy(deg_sh.at[pl.ds(s * RPT, RPT)], stga_v)

    def _repack(i, _):
        for m in range(8):
            stgb_v[i, pl.ds(m * DW, DW)] = stga_v[i * 8 + m, :]
        return _
    lax.fori_loop(0, RPT // 8, _repack, None)
    pltpu.sync_copy(stgb_v, degt_hbm.at[pl.ds(s * (RPT // 8), RPT // 8)])


@functools.partial(
    pl.kernel,
    out_type=_f32((2, NP // 2, D0)),
    mesh=_MESH,
    compiler_params=_SC_PARAMS,
    scratch_types=[
        pltpu.VMEM((80, CE), jnp.int32),
        pltpu.VMEM((80, CE), jnp.int32),
        pltpu.VMEM((16,), jnp.int32),
        pltpu.VMEM((CE, D0), jnp.float32),
        pltpu.VMEM((CE, D0), jnp.float32),
        pltpu.VMEM((CE, D2), jnp.float32),
        pltpu.VMEM((80, D2), jnp.float32),
        pltpu.VMEM((40, D0), jnp.float32),
        pltpu.VMEM_SHARED((NP, D2), jnp.float32),
        pltpu.SemaphoreType.DMA,
        pltpu.SemaphoreType.DMA,
        pltpu.SemaphoreType.DMA,
    ],
)
def _sc_edge_pass(y_hbm, src_hbm, dst_hbm, nch_hbm, aggp_hbm,
                  sidx_v, didx_v, nch_v, rowsa_v, rowsb_v, half_v,
                  stga_v, stgb_v, acc_sh, gsa, gsb, ssm):
    s = lax.axis_index("s")
    w = s

    pltpu.sync_copy(nch_hbm, nch_v)
    nch_vec = nch_v[...]

    def _zero_row(i, _):
        for q in range(D2 // 16):
            stga_v[i, pl.ds(q * 16, 16)] = jnp.zeros((16,), jnp.float32)
        return _

    def _gather(j, rows_v, gsem):
        return pltpu.async_copy(y_hbm.at[sidx_v.at[j]], rows_v, gsem)

    def _gwait(rows_v, gsem):
        pltpu.make_async_copy(y_hbm.at[sidx_v.at[0]], rows_v, gsem).wait()

    def _scat(j):
        pltpu.async_copy(half_v, acc_sh.at[didx_v.at[j]], ssm, add=True)

    def _swait():
        pltpu.make_async_copy(half_v, acc_sh.at[didx_v.at[0]], ssm).wait()

    for h in (0, 1):
        lax.fori_loop(0, 80, _zero_row, None)
        for k in range(8):
            pltpu.sync_copy(stga_v, acc_sh.at[pl.ds(s * RPT + k * 80, 80)])
        plsc.subcore_barrier()

        n_total = nch_vec[h]
        for r, rnd_sz in ((0, 80), (1, KE - 80)):
            nr = jnp.clip(n_total - r * 80, 0, rnd_sz)

            def _extract(rows_v):
                def _row(rr, __):
                    for q in range(D2 // 16):
                        half_v[rr, pl.ds(q * 16, 16)] = (
                            rows_v[rr, pl.ds(h * D2 + q * 16, 16)])
                    return __
                lax.fori_loop(0, CE, _row, None)

            @pl.when(nr > 0)
            def _round():
                pltpu.sync_copy(src_hbm.at[w, pl.ds(r * 80, rnd_sz)],
                                sidx_v.at[pl.ds(0, rnd_sz)])
                pltpu.sync_copy(dst_hbm.at[w, pl.ds(r * 80, rnd_sz)],
                                didx_v.at[pl.ds(0, rnd_sz)])
                _gather(0, rowsa_v, gsa)

                def _pair(jj, _):
                    j0 = jj * 2
                    j1 = j0 + 1
                    _gwait(rowsa_v, gsa)

                    @pl.when(j1 < nr)
                    def _():
                        _gather(j1, rowsb_v, gsb)

                    @pl.when(jj > 0)
                    def _():
                        _swait()
                    _extract(rowsa_v)
                    _scat(j0)

                    @pl.when(j1 < nr)
                    def _():
                        _gwait(rowsb_v, gsb)

                        @pl.when(j0 + 2 < nr)
                        def _():
                            _gather(j0 + 2, rowsa_v, gsa)
                        _swait()
                        _extract(rowsb_v)
                        _scat(j1)
                    return _
                lax.fori_loop(0, (nr + 1) // 2, _pair, None)
                _swait()

        plsc.subcore_barrier()

        for k in range(8):
            pltpu.sync_copy(acc_sh.at[pl.ds(s * RPT + k * 80, 80)], stga_v)

            def _repack(i, _):
                for m in range(2):
                    for q in range(D2 // 16):
                        stgb_v[i, pl.ds(m * D2 + q * 16, 16)] = (
                            stga_v[i * 2 + m, pl.ds(q * 16, 16)])
                return _
            lax.fori_loop(0, 40, _repack, None)
            pltpu.sync_copy(
                stgb_v, aggp_hbm.at[h, pl.ds(s * (RPT // 2) + k * 40, 40)])


@functools.partial(
    pl.kernel,
    out_type=_f32((NW, KL, CE)),
    mesh=_MESH,
    compiler_params=_SC_PARAMS,
    scratch_types=[
        pltpu.VMEM((KL, CE), jnp.int32),
        pltpu.VMEM((KL, CE), jnp.int32),
        pltpu.VMEM((CE, D0), jnp.float32),
        pltpu.VMEM((CE, D0), jnp.float32),
        pltpu.VMEM((CE, D0), jnp.float32),
        pltpu.VMEM((CE, D0), jnp.float32),
        pltpu.VMEM((CE, 16), jnp.float32),
        pltpu.VMEM((KL, CE), jnp.float32),
        pltpu.SemaphoreType.DMA,
        pltpu.SemaphoreType.DMA,
    ],
)
def _sc_decode(z_hbm, si_hbm, di_hbm, res_hbm,
               sidx_v, didx_v, sa_v, da_v, sb_v, db_v, prow_v, out_v,
               gsa, gsb):
    s = lax.axis_index("s")
    w = s

    pltpu.sync_copy(si_hbm.at[w], sidx_v)
    pltpu.sync_copy(di_hbm.at[w], didx_v)

    def _issue(t, srows, drows, gsem):
        pltpu.async_copy(z_hbm.at[sidx_v.at[t]], srows, gsem)
        pltpu.async_copy(z_hbm.at[didx_v.at[t]], drows, gsem)

    def _wait(srows, drows, gsem):
        pltpu.make_async_copy(z_hbm.at[sidx_v.at[0]], srows, gsem).wait()
        pltpu.make_async_copy(z_hbm.at[didx_v.at[0]], drows, gsem).wait()

    def _compute(t, srows, drows):
        def _row(rr, __):
            p = (srows[rr, pl.ds(0, 16)] * drows[rr, pl.ds(0, 16)]
                 + srows[rr, pl.ds(16, 16)] * drows[rr, pl.ds(16, 16)]
                 + srows[rr, pl.ds(32, 16)] * drows[rr, pl.ds(32, 16)]
                 + srows[rr, pl.ds(48, 16)] * drows[rr, pl.ds(48, 16)])
            prow_v[rr, :] = p
            return __
        lax.fori_loop(0, CE, _row, None)
        lane = lax.iota(jnp.int32, 16)
        for g in range(CE // 16):
            erow = lane + g * 16
            acc = jnp.zeros((16,), jnp.float32)
            for q in range(16):
                col = jnp.full((16,), q, jnp.int32)
                acc = acc + plsc.load_gather(prow_v, [erow, col])
            out_v[t, pl.ds(g * 16, 16)] = acc

    _issue(0, sa_v, da_v, gsa)

    def _pair(tt, _):
        t0 = tt * 2
        t1 = t0 + 1
        _wait(sa_v, da_v, gsa)
        _issue(t1, sb_v, db_v, gsb)
        _compute(t0, sa_v, da_v)

        @pl.when(t0 + 2 < KL)
        def _():
            _issue(t0 + 2, sa_v, da_v, gsa)
        _wait(sb_v, db_v, gsb)
        _compute(t1, sb_v, db_v)
        return _
    lax.fori_loop(0, KL // 2, _pair, None)
    pltpu.sync_copy(out_v, res_hbm.at[w])


_TCB = 1280
_TCG = NP // _TCB


def _tc1_body(h0_ref, w1_ref, degt_ref, xw_ref, y_ref, dinv_ref):
    deg = degt_ref[:, 0:1] + 1.0
    dinv = lax.rsqrt(deg)
    xw = jnp.dot(h0_ref[...], w1_ref[...], preferred_element_type=jnp.float32)
    xw_ref[...] = xw
    y_ref[...] = xw * dinv
    dinv_ref[...] = dinv


def _tc2_body(aggp_ref, xw1_ref, dinv_ref, w2_ref, b1_ref,
              xw2_ref, y2_ref):
    dinv = dinv_ref[...]
    agg = jnp.concatenate([aggp_ref[0], aggp_ref[1]], axis=1)
    h1 = jnp.maximum(agg * dinv + xw1_ref[...] * (dinv * dinv)
                     + b1_ref[...][None, :], 0.0)
    xw2 = jnp.dot(h1, w2_ref[...], preferred_element_type=jnp.float32)
    xw2_ref[...] = xw2
    y2 = xw2 * dinv
    y2_ref[...] = jnp.concatenate(
        [y2, jnp.zeros_like(y2)], axis=1)


def _tc3_body(aggp_ref, xw2_ref, dinv_ref, b2_ref, z_ref):
    dinv = dinv_ref[...]
    agg = aggp_ref[0]
    z = (agg * dinv + xw2_ref[...] * (dinv * dinv)
         + b2_ref[...][None, :])
    z_ref[...] = jnp.concatenate(
        [z, jnp.zeros_like(z)], axis=1)


def _tc1(h0, W1, degt):
    return pl.pallas_call(
        _tc1_body,
        grid=(_TCG,),
        in_specs=[
            pl.BlockSpec((_TCB, D0), lambda i: (i, 0)),
            pl.BlockSpec((D0, D0), lambda i: (0, 0)),
            pl.BlockSpec((_TCB, DW), lambda i: (i, 0)),
        ],
        out_specs=[
            pl.BlockSpec((_TCB, D0), lambda i: (i, 0)),
            pl.BlockSpec((_TCB, D0), lambda i: (i, 0)),
            pl.BlockSpec((_TCB, 1), lambda i: (i, 0)),
        ],
        out_shape=[_f32((NP, D0)), _f32((NP, D0)), _f32((NP, 1))],
    )(h0, W1, degt)


def _tc2(aggp1, xw1, dinv, W2, b1):
    return pl.pallas_call(
        _tc2_body,
        grid=(_TCG,),
        in_specs=[
            pl.BlockSpec((2, _TCB, D2), lambda i: (0, i, 0)),
            pl.BlockSpec((_TCB, D0), lambda i: (i, 0)),
            pl.BlockSpec((_TCB, 1), lambda i: (i, 0)),
            pl.BlockSpec((D0, D2), lambda i: (0, 0)),
            pl.BlockSpec((D0,), lambda i: (0,)),
        ],
        out_specs=[
            pl.BlockSpec((_TCB, D2), lambda i: (i, 0)),
            pl.BlockSpec((_TCB, D0), lambda i: (i, 0)),
        ],
        out_shape=[_f32((NP, D2)), _f32((NP, D0))],
    )(aggp1, xw1, dinv, W2, b1)


def _tc3(aggp2, xw2, dinv, b2):
    return pl.pallas_call(
        _tc3_body,
        grid=(_TCG,),
        in_specs=[
            pl.BlockSpec((2, _TCB, D2), lambda i: (0, i, 0)),
            pl.BlockSpec((_TCB, D2), lambda i: (i, 0)),
            pl.BlockSpec((_TCB, 1), lambda i: (i, 0)),
            pl.BlockSpec((D2,), lambda i: (0,)),
        ],
        out_specs=pl.BlockSpec((_TCB, D0), lambda i: (i, 0)),
        out_shape=_f32((NP, D0)),
    )(aggp2, xw2, dinv, b2)


def kernel(x, edge_index, edge_label_index, emb, W1, b1, W2, b2):
    xi = jnp.pad(x[:, 0].astype(jnp.int32), (0, NP - N)).reshape(NW, KX, CE)
    src = jnp.pad(edge_index[0].astype(jnp.int32),
                  (0, E_PAD - E)).reshape(NW, KE, CE)
    dst = jnp.pad(edge_index[1].astype(jnp.int32), (0, E_PAD - E),
                  constant_values=PAD_DST).reshape(NW, KE, CE)
    si = jnp.pad(edge_label_index[0].astype(jnp.int32),
                 (0, NLP_PAD - NLP)).reshape(NW, KL, CE)
    di = jnp.pad(edge_label_index[1].astype(jnp.int32),
                 (0, NLP_PAD - NLP)).reshape(NW, KL, CE)
    nch1 = jnp.full((16,), KE, jnp.int32)
    nch2 = jnp.full((16,), KE, jnp.int32).at[1].set(0)

    h0, degt = _sc_gather_deg(emb, xi, dst)
    degt = degt.reshape(NP, DW)
    xw1, y1, dinv = _tc1(h0, W1, degt)
    aggp1 = _sc_edge_pass(y1, src, dst, nch1).reshape(2, NP, D2)
    xw2, y2 = _tc2(aggp1, xw1, dinv, W2, b1)
    aggp2 = _sc_edge_pass(y2, src, dst, nch2).reshape(2, NP, D2)
    z = _tc3(aggp2, xw2, dinv, b2)
    res = _sc_decode(z, si, di)
    return res.reshape(NLP_PAD)[:NLP]


# speedup vs baseline: 6.4298x; 1.0316x over previous
import functools

import jax
import jax.numpy as jnp
from jax import lax
from jax.experimental import pallas as pl
from jax.experimental.pallas import tpu as pltpu
from jax.experimental.pallas import tpu_sc as plsc

N = 10000
NP = 10240
E = 320000
NLP = 40000
NLP_PAD = 40960
D0 = 128
D2 = 64
NW = 16
CE = 128
KE = 157
E_PAD = NW * KE * CE
KX = NP // NW // CE
KL = NLP_PAD // NW // CE
RPT = NP // NW
DW = 16
PAD_DST = NP - 1

_MESH = plsc.VectorSubcoreMesh(core_axis_name="c", subcore_axis_name="s",
                               num_cores=1)
_SC_PARAMS = pltpu.CompilerParams(use_tc_tiling_on_sc=False,
                                  needs_layout_passes=False)


def _f32(shape):
    return jax.ShapeDtypeStruct(shape, jnp.float32)


@functools.partial(
    pl.kernel,
    out_type=(_f32((NP, D0)), _f32((NP // 8, D0))),
    mesh=_MESH,
    compiler_params=_SC_PARAMS,
    scratch_types=[
        pltpu.VMEM((KX, CE), jnp.int32),
        pltpu.VMEM((KE, CE), jnp.int32),
        pltpu.VMEM((CE, D0), jnp.float32),
        pltpu.VMEM((CE, DW), jnp.float32),
        pltpu.VMEM((RPT, DW), jnp.float32),
        pltpu.VMEM((RPT // 8, D0), jnp.float32),
        pltpu.VMEM_SHARED((NP, DW), jnp.float32),
        pltpu.SemaphoreType.DMA,
    ],
)
def _sc_gather_deg(emb_hbm, xi_hbm, dst_hbm, h0_hbm, degt_hbm,
                   xidx_v, didx_v, rows_v, ones_v, stga_v, stgb_v,
                   deg_sh, sem):
    s = lax.axis_index("s")
    w = s

    def _zero_row(i, _):
        stga_v[i, :] = jnp.zeros((DW,), jnp.float32)
        return _
    lax.fori_loop(0, RPT, _zero_row, None)
    pltpu.sync_copy(stga_v, deg_sh.at[pl.ds(s * RPT, RPT)])

    pltpu.sync_copy(xi_hbm.at[w], xidx_v)
    for k in range(KX):
        pltpu.async_copy(emb_hbm.at[xidx_v.at[k]], rows_v, sem).wait()
        pltpu.sync_copy(rows_v, h0_hbm.at[pl.ds((w * KX + k) * CE, CE)])

    def _ones_row(i, _):
        ones_v[i, :] = jnp.full((DW,), 1.0, jnp.float32)
        return _
    lax.fori_loop(0, CE, _ones_row, None)

    plsc.subcore_barrier()

    pltpu.sync_copy(dst_hbm.at[w], didx_v)

    def _deg_chunk(j, _):
        pltpu.sync_copy(ones_v, deg_sh.at[didx_v.at[j]], add=True)
        return _
    lax.fori_loop(0, KE, _deg_chunk, None)

    plsc.subcore_barrier()

    pltpu.sync_copy(deg_sh.at[pl.ds(s * RPT, RPT)], stga_v)

    def _repack(i, _):
        for m in range(8):
            stgb_v[i, pl.ds(m * DW, DW)] = stga_v[i * 8 + m, :]
        return _
    lax.fori_loop(0, RPT // 8, _repack, None)
    pltpu.sync_copy(stgb_v, degt_hbm.at[pl.ds(s * (RPT // 8), RPT // 8)])


@functools.partial(
    pl.kernel,
    out_type=_f32((2, NP // 2, D0)),
    mesh=_MESH,
    compiler_params=_SC_PARAMS,
    scratch_types=[
        pltpu.VMEM((80, CE), jnp.int32),
        pltpu.VMEM((80, CE), jnp.int32),
        pltpu.VMEM((16,), jnp.int32),
        pltpu.VMEM((CE, D0), jnp.float32),
        pltpu.VMEM((CE, D0), jnp.float32),
        pltpu.VMEM((CE, D2), jnp.float32),
        pltpu.VMEM((80, D2), jnp.float32),
        pltpu.VMEM((40, D0), jnp.float32),
        pltpu.VMEM_SHARED((NP, D2), jnp.float32),
        pltpu.SemaphoreType.DMA,
        pltpu.SemaphoreType.DMA,
        pltpu.SemaphoreType.DMA,
    ],
)
def _sc_edge_pass(y_hbm, src_hbm, dst_hbm, nch_hbm, aggp_hbm,
                  sidx_v, didx_v, nch_v, rowsa_v, rowsb_v, half_v,
                  stga_v, stgb_v, acc_sh, gsa, gsb, ssm):
    s = lax.axis_index("s")
    w = s

    pltpu.sync_copy(nch_hbm, nch_v)
    nch_vec = nch_v[...]

    def _zero_row(i, _):
        for q in range(D2 // 16):
            stga_v[i, pl.ds(q * 16, 16)] = jnp.zeros((16,), jnp.float32)
        return _

    def _gather(j, rows_v, gsem):
        return pltpu.async_copy(y_hbm.at[sidx_v.at[j]], rows_v, gsem)

    def _gwait(rows_v, gsem):
        pltpu.make_async_copy(y_hbm.at[sidx_v.at[0]], rows_v, gsem).wait()

    def _scat(j):
        pltpu.async_copy(half_v, acc_sh.at[didx_v.at[j]], ssm, add=True)

    def _swait():
        pltpu.make_async_copy(half_v, acc_sh.at[didx_v.at[0]], ssm).wait()

    for h in (0, 1):
        lax.fori_loop(0, 80, _zero_row, None)
        for k in range(8):
            pltpu.sync_copy(stga_v, acc_sh.at[pl.ds(s * RPT + k * 80, 80)])
        plsc.subcore_barrier()

        n_total = nch_vec[h]
        for r, rnd_sz in ((0, 80), (1, KE - 80)):
            nr = jnp.clip(n_total - r * 80, 0, rnd_sz)

            def _extract(rows_v):
                def _row4(ii, __):
                    for u in range(4):
                        rr = ii * 4 + u
                        for q in range(D2 // 16):
                            half_v[rr, pl.ds(q * 16, 16)] = (
                                rows_v[rr, pl.ds(h * D2 + q * 16, 16)])
                    return __
                lax.fori_loop(0, CE // 4, _row4, None)

            @pl.when(nr > 0)
            def _round():
                pltpu.sync_copy(src_hbm.at[w, pl.ds(r * 80, rnd_sz)],
                                sidx_v.at[pl.ds(0, rnd_sz)])
                pltpu.sync_copy(dst_hbm.at[w, pl.ds(r * 80, rnd_sz)],
                                didx_v.at[pl.ds(0, rnd_sz)])
                _gather(0, rowsa_v, gsa)

                def _pair(jj, _):
                    j0 = jj * 2
                    j1 = j0 + 1
                    _gwait(rowsa_v, gsa)

                    @pl.when(j1 < nr)
                    def _():
                        _gather(j1, rowsb_v, gsb)

                    @pl.when(jj > 0)
                    def _():
                        _swait()
                    _extract(rowsa_v)
                    _scat(j0)

                    @pl.when(j1 < nr)
                    def _():
                        _gwait(rowsb_v, gsb)

                        @pl.when(j0 + 2 < nr)
                        def _():
                            _gather(j0 + 2, rowsa_v, gsa)
                        _swait()
                        _extract(rowsb_v)
                        _scat(j1)
                    return _
                lax.fori_loop(0, (nr + 1) // 2, _pair, None)
                _swait()

        plsc.subcore_barrier()

        for k in range(8):
            pltpu.sync_copy(acc_sh.at[pl.ds(s * RPT + k * 80, 80)], stga_v)

            def _repack(i, _):
                for m in range(2):
                    for q in range(D2 // 16):
                        stgb_v[i, pl.ds(m * D2 + q * 16, 16)] = (
                            stga_v[i * 2 + m, pl.ds(q * 16, 16)])
                return _
            lax.fori_loop(0, 40, _repack, None)
            pltpu.sync_copy(
                stgb_v, aggp_hbm.at[h, pl.ds(s * (RPT // 2) + k * 40, 40)])


@functools.partial(
    pl.kernel,
    out_type=_f32((NW, KL, CE)),
    mesh=_MESH,
    compiler_params=_SC_PARAMS,
    scratch_types=[
        pltpu.VMEM((KL, CE), jnp.int32),
        pltpu.VMEM((KL, CE), jnp.int32),
        pltpu.VMEM((CE, D0), jnp.float32),
        pltpu.VMEM((CE, D0), jnp.float32),
        pltpu.VMEM((CE, D0), jnp.float32),
        pltpu.VMEM((CE, D0), jnp.float32),
        pltpu.VMEM((CE, 16), jnp.float32),
        pltpu.VMEM((KL, CE), jnp.float32),
        pltpu.SemaphoreType.DMA,
        pltpu.SemaphoreType.DMA,
    ],
)
def _sc_decode(z_hbm, si_hbm, di_hbm, res_hbm,
               sidx_v, didx_v, sa_v, da_v, sb_v, db_v, prow_v, out_v,
               gsa, gsb):
    s = lax.axis_index("s")
    w = s

    pltpu.sync_copy(si_hbm.at[w], sidx_v)
    pltpu.sync_copy(di_hbm.at[w], didx_v)

    def _issue(t, srows, drows, gsem):
        pltpu.async_copy(z_hbm.at[sidx_v.at[t]], srows, gsem)
        pltpu.async_copy(z_hbm.at[didx_v.at[t]], drows, gsem)

    def _wait(srows, drows, gsem):
        pltpu.make_async_copy(z_hbm.at[sidx_v.at[0]], srows, gsem).wait()
        pltpu.make_async_copy(z_hbm.at[didx_v.at[0]], drows, gsem).wait()

    def _compute(t, srows, drows):
        def _row(rr, __):
            p = (srows[rr, pl.ds(0, 16)] * drows[rr, pl.ds(0, 16)]
                 + srows[rr, pl.ds(16, 16)] * drows[rr, pl.ds(16, 16)]
                 + srows[rr, pl.ds(32, 16)] * drows[rr, pl.ds(32, 16)]
                 + srows[rr, pl.ds(48, 16)] * drows[rr, pl.ds(48, 16)])
            prow_v[rr, :] = p
            return __
        lax.fori_loop(0, CE, _row, None)
        lane = lax.iota(jnp.int32, 16)
        for g in range(CE // 16):
            erow = lane + g * 16
            acc = jnp.zeros((16,), jnp.float32)
            for q in range(16):
                col = jnp.full((16,), q, jnp.int32)
                acc = acc + plsc.load_gather(prow_v, [erow, col])
            out_v[t, pl.ds(g * 16, 16)] = acc

    _issue(0, sa_v, da_v, gsa)

    def _pair(tt, _):
        t0 = tt * 2
        t1 = t0 + 1
        _wait(sa_v, da_v, gsa)
        _issue(t1, sb_v, db_v, gsb)
        _compute(t0, sa_v, da_v)

        @pl.when(t0 + 2 < KL)
        def _():
            _issue(t0 + 2, sa_v, da_v, gsa)
        _wait(sb_v, db_v, gsb)
        _compute(t1, sb_v, db_v)
        return _
    lax.fori_loop(0, KL // 2, _pair, None)
    pltpu.sync_copy(out_v, res_hbm.at[w])


_TCB = 1280
_TCG = NP // _TCB


def _tc1_body(h0_ref, w1_ref, degt_ref, xw_ref, y_ref, dinv_ref):
    deg = degt_ref[:, 0:1] + 1.0
    dinv = lax.rsqrt(deg)
    xw = jnp.dot(h0_ref[...], w1_ref[...], preferred_element_type=jnp.float32)
    xw_ref[...] = xw
    y_ref[...] = xw * dinv
    dinv_ref[...] = dinv


def _tc2_body(aggp_ref, xw1_ref, dinv_ref, w2_ref, b1_ref,
              xw2_ref, y2_ref):
    dinv = dinv_ref[...]
    agg = jnp.concatenate([aggp_ref[0], aggp_ref[1]], axis=1)
    h1 = jnp.maximum(agg * dinv + xw1_ref[...] * (dinv * dinv)
                     + b1_ref[...][None, :], 0.0)
    xw2 = jnp.dot(h1, w2_ref[...], preferred_element_type=jnp.float32)
    xw2_ref[...] = xw2
    y2 = xw2 * dinv
    y2_ref[...] = jnp.concatenate(
        [y2, jnp.zeros_like(y2)], axis=1)


def _tc3_body(aggp_ref, xw2_ref, dinv_ref, b2_ref, z_ref):
    dinv = dinv_ref[...]
    agg = aggp_ref[0]
    z = (agg * dinv + xw2_ref[...] * (dinv * dinv)
         + b2_ref[...][None, :])
    z_ref[...] = jnp.concatenate(
        [z, jnp.zeros_like(z)], axis=1)


def _tc1(h0, W1, degt):
    return pl.pallas_call(
        _tc1_body,
        grid=(_TCG,),
        in_specs=[
            pl.BlockSpec((_TCB, D0), lambda i: (i, 0)),
            pl.BlockSpec((D0, D0), lambda i: (0, 0)),
            pl.BlockSpec((_TCB, DW), lambda i: (i, 0)),
        ],
        out_specs=[
            pl.BlockSpec((_TCB, D0), lambda i: (i, 0)),
            pl.BlockSpec((_TCB, D0), lambda i: (i, 0)),
            pl.BlockSpec((_TCB, 1), lambda i: (i, 0)),
        ],
        out_shape=[_f32((NP, D0)), _f32((NP, D0)), _f32((NP, 1))],
    )(h0, W1, degt)


def _tc2(aggp1, xw1, dinv, W2, b1):
    return pl.pallas_call(
        _tc2_body,
        grid=(_TCG,),
        in_specs=[
            pl.BlockSpec((2, _TCB, D2), lambda i: (0, i, 0)),
            pl.BlockSpec((_TCB, D0), lambda i: (i, 0)),
            pl.BlockSpec((_TCB, 1), lambda i: (i, 0)),
            pl.BlockSpec((D0, D2), lambda i: (0, 0)),
            pl.BlockSpec((D0,), lambda i: (0,)),
        ],
        out_specs=[
            pl.BlockSpec((_TCB, D2), lambda i: (i, 0)),
            pl.BlockSpec((_TCB, D0), lambda i: (i, 0)),
        ],
        out_shape=[_f32((NP, D2)), _f32((NP, D0))],
    )(aggp1, xw1, dinv, W2, b1)


def _tc3(aggp2, xw2, dinv, b2):
    return pl.pallas_call(
        _tc3_body,
        grid=(_TCG,),
        in_specs=[
            pl.BlockSpec((2, _TCB, D2), lambda i: (0, i, 0)),
            pl.BlockSpec((_TCB, D2), lambda i: (i, 0)),
            pl.BlockSpec((_TCB, 1), lambda i: (i, 0)),
            pl.BlockSpec((D2,), lambda i: (0,)),
        ],
        out_specs=pl.BlockSpec((_TCB, D0), lambda i: (i, 0)),
        out_shape=_f32((NP, D0)),
    )(aggp2, xw2, dinv, b2)


def kernel(x, edge_index, edge_label_index, emb, W1, b1, W2, b2):
    xi = jnp.pad(x[:, 0].astype(jnp.int32), (0, NP - N)).reshape(NW, KX, CE)
    src = jnp.pad(edge_index[0].astype(jnp.int32),
                  (0, E_PAD - E)).reshape(NW, KE, CE)
    dst = jnp.pad(edge_index[1].astype(jnp.int32), (0, E_PAD - E),
                  constant_values=PAD_DST).reshape(NW, KE, CE)
    si = jnp.pad(edge_label_index[0].astype(jnp.int32),
                 (0, NLP_PAD - NLP)).reshape(NW, KL, CE)
    di = jnp.pad(edge_label_index[1].astype(jnp.int32),
                 (0, NLP_PAD - NLP)).reshape(NW, KL, CE)
    nch1 = jnp.full((16,), KE, jnp.int32)
    nch2 = jnp.full((16,), KE, jnp.int32).at[1].set(0)

    h0, degt = _sc_gather_deg(emb, xi, dst)
    degt = degt.reshape(NP, DW)
    xw1, y1, dinv = _tc1(h0, W1, degt)
    aggp1 = _sc_edge_pass(y1, src, dst, nch1).reshape(2, NP, D2)
    xw2, y2 = _tc2(aggp1, xw1, dinv, W2, b1)
    aggp2 = _sc_edge_pass(y2, src, dst, nch2).reshape(2, NP, D2)
    z = _tc3(aggp2, xw2, dinv, b2)
    res = _sc_decode(z, si, di)
    return res.reshape(NLP_PAD)[:NLP]


# speedup vs baseline: 8.0524x; 1.2524x over previous
import functools

import jax
import jax.numpy as jnp
from jax import lax
from jax.experimental import pallas as pl
from jax.experimental.pallas import tpu as pltpu
from jax.experimental.pallas import tpu_sc as plsc

N = 10000
NP = 10240
E = 320000
NLP = 40000
NLP_PAD = 40960
D0 = 128
D2 = 64
NW = 32
CE = 128
KE = 79
E_PAD = NW * KE * CE
NXC = NP // CE
KL = NLP_PAD // NW // CE
RPT = NP // NW
DW = 16
PAD_DST = NP - 1

_MESH = plsc.VectorSubcoreMesh(core_axis_name="c", subcore_axis_name="s",
                               num_cores=2)
_SC_PARAMS = pltpu.CompilerParams(use_tc_tiling_on_sc=False,
                                  needs_layout_passes=False)


def _f32(shape):
    return jax.ShapeDtypeStruct(shape, jnp.float32)


@functools.partial(
    pl.kernel,
    out_type=(_f32((NP, D0)), _f32((2, NP // 8, D0))),
    mesh=_MESH,
    compiler_params=_SC_PARAMS,
    scratch_types=[
        pltpu.VMEM((CE,), jnp.int32),
        pltpu.VMEM((KE, CE), jnp.int32),
        pltpu.VMEM((CE, D0), jnp.float32),
        pltpu.VMEM((CE, DW), jnp.float32),
        pltpu.VMEM((RPT, DW), jnp.float32),
        pltpu.VMEM((RPT // 8, D0), jnp.float32),
        pltpu.VMEM_SHARED((NP, DW), jnp.float32),
        pltpu.SemaphoreType.DMA,
    ],
)
def _sc_gather_deg(emb_hbm, xi_hbm, dst_hbm, h0_hbm, degp_hbm,
                   xidx_v, didx_v, rows_v, ones_v, stga_v, stgb_v,
                   deg_sh, sem):
    c = lax.axis_index("c")
    s = lax.axis_index("s")
    w = s * 2 + c

    def _zero_row(i, _):
        stga_v[i, :] = jnp.zeros((DW,), jnp.float32)
        return _
    lax.fori_loop(0, RPT, _zero_row, None)
    pltpu.sync_copy(stga_v, deg_sh.at[pl.ds(s * RPT, RPT)])

    for k in range(3):
        cid = w + NW * k
        if (NW * k) < NXC:
            @pl.when(cid < NXC)
            def _():
                pltpu.sync_copy(xi_hbm.at[cid], xidx_v)
                pltpu.async_copy(emb_hbm.at[xidx_v], rows_v, sem).wait()
                pltpu.sync_copy(rows_v, h0_hbm.at[pl.ds(cid * CE, CE)])

    def _ones_row(i, _):
        ones_v[i, :] = jnp.full((DW,), 1.0, jnp.float32)
        return _
    lax.fori_loop(0, CE, _ones_row, None)

    plsc.subcore_barrier()

    pltpu.sync_copy(dst_hbm.at[w], didx_v)

    def _deg_chunk(j, _):
        pltpu.sync_copy(ones_v, deg_sh.at[didx_v.at[j]], add=True)
        return _
    lax.fori_loop(0, KE, _deg_chunk, None)

    plsc.subcore_barrier()

    pltpu.sync_copy(deg_sh.at[pl.ds(s * RPT, RPT)], stga_v)

    def _repack(i, _):
        for m in range(8):
            stgb_v[i, pl.ds(m * DW, DW)] = stga_v[i * 8 + m, :]
        return _
    lax.fori_loop(0, RPT // 8, _repack, None)
    pltpu.sync_copy(stgb_v,
                    degp_hbm.at[c, pl.ds(s * (RPT // 8), RPT // 8)])


@functools.partial(
    pl.kernel,
    out_type=_f32((2, 2, NP // 2, D0)),
    mesh=_MESH,
    compiler_params=_SC_PARAMS,
    scratch_types=[
        pltpu.VMEM((KE, CE), jnp.int32),
        pltpu.VMEM((KE, CE), jnp.int32),
        pltpu.VMEM((16,), jnp.int32),
        pltpu.VMEM((CE, D0), jnp.float32),
        pltpu.VMEM((CE, D0), jnp.float32),
        pltpu.VMEM((CE, D2), jnp.float32),
        pltpu.VMEM((80, D2), jnp.float32),
        pltpu.VMEM((40, D0), jnp.float32),
        pltpu.VMEM_SHARED((NP, D2), jnp.float32),
        pltpu.SemaphoreType.DMA,
        pltpu.SemaphoreType.DMA,
        pltpu.SemaphoreType.DMA,
    ],
)
def _sc_edge_pass(y_hbm, src_hbm, dst_hbm, nch_hbm, aggp_hbm,
                  sidx_v, didx_v, nch_v, rowsa_v, rowsb_v, half_v,
                  stga_v, stgb_v, acc_sh, gsa, gsb, ssm):
    c = lax.axis_index("c")
    s = lax.axis_index("s")
    w = s * 2 + c

    pltpu.sync_copy(nch_hbm, nch_v)
    nch_vec = nch_v[...]

    def _zero_row(i, _):
        for q in range(D2 // 16):
            stga_v[i, pl.ds(q * 16, 16)] = jnp.zeros((16,), jnp.float32)
        return _

    def _gather(j, rows_v, gsem):
        return pltpu.async_copy(y_hbm.at[sidx_v.at[j]], rows_v, gsem)

    def _gwait(rows_v, gsem):
        pltpu.make_async_copy(y_hbm.at[sidx_v.at[0]], rows_v, gsem).wait()

    def _scat(j):
        pltpu.async_copy(half_v, acc_sh.at[didx_v.at[j]], ssm, add=True)

    def _swait():
        pltpu.make_async_copy(half_v, acc_sh.at[didx_v.at[0]], ssm).wait()

    for h in (0, 1):
        lax.fori_loop(0, 80, _zero_row, None)
        for k in range(8):
            pltpu.sync_copy(stga_v, acc_sh.at[pl.ds(s * RPT + k * 80, 80)])
        plsc.subcore_barrier()

        n_total = nch_vec[h]
        for r, rnd_sz in ((0, KE),):
            nr = jnp.clip(n_total, 0, rnd_sz)

            def _extract(rows_v):
                def _row4(ii, __):
                    for u in range(4):
                        rr = ii * 4 + u
                        for q in range(D2 // 16):
                            half_v[rr, pl.ds(q * 16, 16)] = (
                                rows_v[rr, pl.ds(h * D2 + q * 16, 16)])
                    return __
                lax.fori_loop(0, CE // 4, _row4, None)

            @pl.when(nr > 0)
            def _round():
                pltpu.sync_copy(src_hbm.at[w], sidx_v)
                pltpu.sync_copy(dst_hbm.at[w], didx_v)
                _gather(0, rowsa_v, gsa)

                def _pair(jj, _):
                    j0 = jj * 2
                    j1 = j0 + 1
                    _gwait(rowsa_v, gsa)

                    @pl.when(j1 < nr)
                    def _():
                        _gather(j1, rowsb_v, gsb)

                    @pl.when(jj > 0)
                    def _():
                        _swait()
                    _extract(rowsa_v)
                    _scat(j0)

                    @pl.when(j1 < nr)
                    def _():
                        _gwait(rowsb_v, gsb)

                        @pl.when(j0 + 2 < nr)
                        def _():
                            _gather(j0 + 2, rowsa_v, gsa)
                        _swait()
                        _extract(rowsb_v)
                        _scat(j1)
                    return _
                lax.fori_loop(0, (nr + 1) // 2, _pair, None)
                _swait()

        plsc.subcore_barrier()

        for k in range(8):
            pltpu.sync_copy(acc_sh.at[pl.ds(s * RPT + k * 80, 80)], stga_v)

            def _repack(i, _):
                for m in range(2):
                    for q in range(D2 // 16):
                        stgb_v[i, pl.ds(m * D2 + q * 16, 16)] = (
                            stga_v[i * 2 + m, pl.ds(q * 16, 16)])
                return _
            lax.fori_loop(0, 40, _repack, None)
            pltpu.sync_copy(
                stgb_v, aggp_hbm.at[c, h, pl.ds(s * (RPT // 2) + k * 40, 40)])


@functools.partial(
    pl.kernel,
    out_type=_f32((NW, KL, CE)),
    mesh=_MESH,
    compiler_params=_SC_PARAMS,
    scratch_types=[
        pltpu.VMEM((KL, CE), jnp.int32),
        pltpu.VMEM((KL, CE), jnp.int32),
        pltpu.VMEM((CE, D0), jnp.float32),
        pltpu.VMEM((CE, D0), jnp.float32),
        pltpu.VMEM((CE, D0), jnp.float32),
        pltpu.VMEM((CE, D0), jnp.float32),
        pltpu.VMEM((CE, 16), jnp.float32),
        pltpu.VMEM((KL, CE), jnp.float32),
        pltpu.SemaphoreType.DMA,
        pltpu.SemaphoreType.DMA,
    ],
)
def _sc_decode(z_hbm, si_hbm, di_hbm, res_hbm,
               sidx_v, didx_v, sa_v, da_v, sb_v, db_v, prow_v, out_v,
               gsa, gsb):
    c = lax.axis_index("c")
    s = lax.axis_index("s")
    w = s * 2 + c

    pltpu.sync_copy(si_hbm.at[w], sidx_v)
    pltpu.sync_copy(di_hbm.at[w], didx_v)

    def _issue(t, srows, drows, gsem):
        pltpu.async_copy(z_hbm.at[sidx_v.at[t]], srows, gsem)
        pltpu.async_copy(z_hbm.at[didx_v.at[t]], drows, gsem)

    def _wait(srows, drows, gsem):
        pltpu.make_async_copy(z_hbm.at[sidx_v.at[0]], srows, gsem).wait()
        pltpu.make_async_copy(z_hbm.at[didx_v.at[0]], drows, gsem).wait()

    def _compute(t, srows, drows):
        def _row(rr, __):
            p = (srows[rr, pl.ds(0, 16)] * drows[rr, pl.ds(0, 16)]
                 + srows[rr, pl.ds(16, 16)] * drows[rr, pl.ds(16, 16)]
                 + srows[rr, pl.ds(32, 16)] * drows[rr, pl.ds(32, 16)]
                 + srows[rr, pl.ds(48, 16)] * drows[rr, pl.ds(48, 16)])
            prow_v[rr, :] = p
            return __
        lax.fori_loop(0, CE, _row, None)
        lane = lax.iota(jnp.int32, 16)
        for g in range(CE // 16):
            erow = lane + g * 16
            acc = jnp.zeros((16,), jnp.float32)
            for q in range(16):
                col = jnp.full((16,), q, jnp.int32)
                acc = acc + plsc.load_gather(prow_v, [erow, col])
            out_v[t, pl.ds(g * 16, 16)] = acc

    _issue(0, sa_v, da_v, gsa)

    def _pair(tt, _):
        t0 = tt * 2
        t1 = t0 + 1
        _wait(sa_v, da_v, gsa)
        _issue(t1, sb_v, db_v, gsb)
        _compute(t0, sa_v, da_v)

        @pl.when(t0 + 2 < KL)
        def _():
            _issue(t0 + 2, sa_v, da_v, gsa)
        _wait(sb_v, db_v, gsb)
        _compute(t1, sb_v, db_v)
        return _
    lax.fori_loop(0, KL // 2, _pair, None)
    pltpu.sync_copy(out_v, res_hbm.at[w])


_TCB = 1280
_TCG = NP // _TCB


def _tc1_body(h0_ref, w1_ref, degt_ref, xw_ref, y_ref, dinv_ref):
    deg = degt_ref[0, :, 0:1] + degt_ref[1, :, 0:1] + 1.0
    dinv = lax.rsqrt(deg)
    xw = jnp.dot(h0_ref[...], w1_ref[...], preferred_element_type=jnp.float32)
    xw_ref[...] = xw
    y_ref[...] = xw * dinv
    dinv_ref[...] = dinv


def _tc2_body(aggp_ref, xw1_ref, dinv_ref, w2_ref, b1_ref,
              xw2_ref, y2_ref):
    dinv = dinv_ref[...]
    agg = jnp.concatenate([aggp_ref[0, 0] + aggp_ref[1, 0],
                           aggp_ref[0, 1] + aggp_ref[1, 1]], axis=1)
    h1 = jnp.maximum(agg * dinv + xw1_ref[...] * (dinv * dinv)
                     + b1_ref[...][None, :], 0.0)
    xw2 = jnp.dot(h1, w2_ref[...], preferred_element_type=jnp.float32)
    xw2_ref[...] = xw2
    y2 = xw2 * dinv
    y2_ref[...] = jnp.concatenate(
        [y2, jnp.zeros_like(y2)], axis=1)


def _tc3_body(aggp_ref, xw2_ref, dinv_ref, b2_ref, z_ref):
    dinv = dinv_ref[...]
    agg = aggp_ref[0, 0] + aggp_ref[1, 0]
    z = (agg * dinv + xw2_ref[...] * (dinv * dinv)
         + b2_ref[...][None, :])
    z_ref[...] = jnp.concatenate(
        [z, jnp.zeros_like(z)], axis=1)


def _tc1(h0, W1, degt):
    return pl.pallas_call(
        _tc1_body,
        grid=(_TCG,),
        in_specs=[
            pl.BlockSpec((_TCB, D0), lambda i: (i, 0)),
            pl.BlockSpec((D0, D0), lambda i: (0, 0)),
            pl.BlockSpec((2, _TCB, DW), lambda i: (0, i, 0)),
        ],
        out_specs=[
            pl.BlockSpec((_TCB, D0), lambda i: (i, 0)),
            pl.BlockSpec((_TCB, D0), lambda i: (i, 0)),
            pl.BlockSpec((_TCB, 1), lambda i: (i, 0)),
        ],
        out_shape=[_f32((NP, D0)), _f32((NP, D0)), _f32((NP, 1))],
    )(h0, W1, degt)


def _tc2(aggp1, xw1, dinv, W2, b1):
    return pl.pallas_call(
        _tc2_body,
        grid=(_TCG,),
        in_specs=[
            pl.BlockSpec((2, 2, _TCB, D2), lambda i: (0, 0, i, 0)),
            pl.BlockSpec((_TCB, D0), lambda i: (i, 0)),
            pl.BlockSpec((_TCB, 1), lambda i: (i, 0)),
            pl.BlockSpec((D0, D2), lambda i: (0, 0)),
            pl.BlockSpec((D0,), lambda i: (0,)),
        ],
        out_specs=[
            pl.BlockSpec((_TCB, D2), lambda i: (i, 0)),
            pl.BlockSpec((_TCB, D0), lambda i: (i, 0)),
        ],
        out_shape=[_f32((NP, D2)), _f32((NP, D0))],
    )(aggp1, xw1, dinv, W2, b1)


def _tc3(aggp2, xw2, dinv, b2):
    return pl.pallas_call(
        _tc3_body,
        grid=(_TCG,),
        in_specs=[
            pl.BlockSpec((2, 2, _TCB, D2), lambda i: (0, 0, i, 0)),
            pl.BlockSpec((_TCB, D2), lambda i: (i, 0)),
            pl.BlockSpec((_TCB, 1), lambda i: (i, 0)),
            pl.BlockSpec((D2,), lambda i: (0,)),
        ],
        out_specs=pl.BlockSpec((_TCB, D0), lambda i: (i, 0)),
        out_shape=_f32((NP, D0)),
    )(aggp2, xw2, dinv, b2)


def kernel(x, edge_index, edge_label_index, emb, W1, b1, W2, b2):
    xi = jnp.pad(x[:, 0].astype(jnp.int32), (0, NP - N)).reshape(NXC, CE)
    src = jnp.pad(edge_index[0].astype(jnp.int32),
                  (0, E_PAD - E)).reshape(NW, KE, CE)
    dst = jnp.pad(edge_index[1].astype(jnp.int32), (0, E_PAD - E),
                  constant_values=PAD_DST).reshape(NW, KE, CE)
    si = jnp.pad(edge_label_index[0].astype(jnp.int32),
                 (0, NLP_PAD - NLP)).reshape(NW, KL, CE)
    di = jnp.pad(edge_label_index[1].astype(jnp.int32),
                 (0, NLP_PAD - NLP)).reshape(NW, KL, CE)
    nch1 = jnp.full((16,), KE, jnp.int32)
    nch2 = jnp.full((16,), KE, jnp.int32).at[1].set(0)

    h0, degp = _sc_gather_deg(emb, xi, dst)
    degp = degp.reshape(2, NP, DW)
    xw1, y1, dinv = _tc1(h0, W1, degp)
    aggp1 = _sc_edge_pass(y1, src, dst, nch1).reshape(2, 2, NP, D2)
    xw2, y2 = _tc2(aggp1, xw1, dinv, W2, b1)
    aggp2 = _sc_edge_pass(y2, src, dst, nch2).reshape(2, 2, NP, D2)
    z = _tc3(aggp2, xw2, dinv, b2)
    res = _sc_decode(z, si, di)
    return res.reshape(NLP_PAD)[:NLP]
